# baseline (speedup 1.0000x reference)
import functools

import jax
import jax.numpy as jnp
from jax import lax
from jax.experimental import pallas as pl
from jax.experimental.pallas import tpu as pltpu
from jax.experimental.pallas import tpu_sc as plsc

S = 8192
B = 2
D = 1024
L = 16
CR = 32


def _make_sc_kernel():
    info = plsc.get_sparse_core_info()
    nc, ns = info.num_cores, info.num_subcores
    nw = nc * ns
    rows_per_w = S // nw
    n_chunks = rows_per_w // CR

    mesh = plsc.VectorSubcoreMesh(core_axis_name="c", subcore_axis_name="s")

    @functools.partial(
        pl.kernel,
        mesh=mesh,
        out_type=jax.ShapeDtypeStruct((S, B, D), jnp.float32),
        scratch_types=[
            pltpu.VMEM((CR, B, D), jnp.float32),
            pltpu.VMEM((CR, D), jnp.float32),
        ],
    )
    def k(x_hbm, emb_hbm, out_hbm, xbuf, ebuf):
        wid = lax.axis_index("s") * nc + lax.axis_index("c")
        base = wid * rows_per_w

        def chunk_body(i, carry):
            r0 = base + i * CR
            pltpu.sync_copy(x_hbm.at[pl.ds(r0, CR)], xbuf)
            pltpu.sync_copy(emb_hbm.at[pl.ds(r0, CR)], ebuf)

            def row_body(r, c2):
                for j in range(D // L):
                    e = ebuf[r, pl.ds(j * L, L)]
                    xbuf[r, 0, pl.ds(j * L, L)] = xbuf[r, 0, pl.ds(j * L, L)] + e
                    xbuf[r, 1, pl.ds(j * L, L)] = xbuf[r, 1, pl.ds(j * L, L)] + e
                return c2

            lax.fori_loop(0, CR, row_body, 0)
            pltpu.sync_copy(xbuf, out_hbm.at[pl.ds(r0, CR)])
            return carry

        lax.fori_loop(0, n_chunks, chunk_body, 0)

    return k


_sc_kernel = _make_sc_kernel()


def kernel(x, emb_weight):
    x3 = x.reshape(S, B, D)
    out = _sc_kernel(x3, emb_weight)
    return out.reshape(S, B, 1, D)
